# baseline (speedup 1.0000x reference)
import jax
import jax.numpy as jnp
from jax import lax
from jax.experimental import pallas as pl
from jax.experimental.pallas import tpu as pltpu
from jax.experimental.pallas import tpu_sc as plsc

N = 16384
D = 768
C = 1000
L = 16
NJ = D // L
NC = 2
NS = 16
NW = NC * NS
CPT = 32
B = 32
EPS = 1e-5


def _recip(x):
    bits = lax.bitcast_convert_type(x, jnp.int32)
    r = lax.bitcast_convert_type(jnp.int32(0x7EF311C3) - bits, jnp.float32)
    for _ in range(4):
        r = r * (2.0 - x * r)
    return r


def _ln_body(x_ref, g_ref, b_ref, y_ref):
    x = x_ref[...]
    ones = jnp.ones((D, 128), jnp.float32)
    dn = (((1,), (0,)), ((), ()))
    s = lax.dot_general(x, ones, dn, precision=lax.Precision.HIGHEST,
                        preferred_element_type=jnp.float32)
    q = lax.dot_general(x * x, ones, dn, precision=lax.Precision.HIGHEST,
                        preferred_element_type=jnp.float32)
    mean = s[:, :1] * (1.0 / D)
    var = q[:, :1] * (1.0 / D) - mean * mean
    y_ref[...] = ((x - mean) * lax.rsqrt(var + EPS) * g_ref[...]
                  + b_ref[...])


def _sc_body(y, ids, ptab, out, ids_v, xb, acc, bnd, prow, orow, sem):
    wid = lax.axis_index("s") * NC + lax.axis_index("c")
    c0 = (wid * CPT).astype(jnp.int32)

    pltpu.sync_copy(ids, ids_v)

    def ids_at(g):
        base = g & ~(L - 1)
        v = ids_v[pl.ds(base, L)]
        off = g - base
        s = v[0]
        for k in range(1, L):
            s = jnp.where(off == k, v[k], s)
        return s

    NG = N // L

    def lower_bound(tgt):
        glo = jnp.int32(0)
        for sbit in range(10, -1, -1):
            cand = glo + jnp.int32(1 << sbit)
            probe = ids_v[pl.ds(jnp.minimum(cand, NG) * L - L, L)][L - 1]
            glo = jnp.where((cand <= NG) & (probe < tgt), cand, glo)
        base = jnp.minimum(glo, NG - 1) * L
        v = ids_v[pl.ds(base, L)]
        cnt = jnp.int32(0)
        for k in range(L):
            cnt = cnt + jnp.where(v[k] < tgt, 1, 0)
        return jnp.where(glo >= NG, jnp.int32(N), glo * L + cnt)

    zero16 = jnp.zeros((L,), jnp.float32)
    zeros48 = tuple(zero16 for _ in range(NJ))

    r0 = lower_bound(c0)
    r1 = lower_bound(c0 + CPT)
    bnd[0] = r0
    bnd[CPT] = r1
    start0 = r0 & ~7
    nblk = jnp.where(r1 > r0, (r1 - start0 + (B - 1)) >> 5, 0)

    def blk_base(k):
        return pl.multiple_of(jnp.minimum(start0 + k * B, N - B), 8)

    @pl.when(nblk > 0)
    def _():
        pltpu.async_copy(y.at[pl.ds(blk_base(0), B), :], xb.at[0], sem)

    def bnd_body(k, carry):
        bnd[k] = lower_bound(c0 + k)
        return carry

    lax.fori_loop(1, CPT, bnd_body, 0)

    def blk_body(k, accs_in):
        p = k & 1
        logical = start0 + k * B
        base = blk_base(k)
        pltpu.make_async_copy(y.at[pl.ds(base, B), :], xb.at[p], sem).wait()

        @pl.when(k + 1 < nblk)
        def _():
            pltpu.async_copy(
                y.at[pl.ds(blk_base(k + 1), B), :], xb.at[(k + 1) & 1], sem)

        i_lo = jnp.maximum(r0, logical) - base
        i_hi = jnp.minimum(r1, base + B) - base
        lc_first = ids_at(base + i_lo) - c0
        lc_last = ids_at(base + i_hi - 1) - c0

        def seg_body(lc, accs):
            first = lc == lc_first
            s_seg = jnp.maximum(bnd[lc], base + i_lo) - base
            e_seg = jnp.minimum(bnd[lc + 1], base + i_hi) - base
            init = tuple(
                jnp.where(first, accs[j], zero16) for j in range(NJ))

            npairs = (e_seg - s_seg) >> 1

            def pair_body(t, a):
                i = s_seg + t * 2
                return tuple(
                    a[j] + (xb[p, i, pl.ds(j * L, L)]
                            + xb[p, i + 1, pl.ds(j * L, L)])
                    for j in range(NJ))

            res = lax.fori_loop(0, npairs, pair_body, init)

            def row_body(i, a):
                return tuple(
                    a[j] + xb[p, i, pl.ds(j * L, L)] for j in range(NJ))

            res = lax.fori_loop(s_seg + npairs * 2, e_seg, row_body, res)
            closed = bnd[lc + 1] <= base + i_hi

            @pl.when(closed)
            def _():
                for j in range(NJ):
                    acc[lc, pl.ds(j * L, L)] = res[j]

            return tuple(
                jnp.where(closed, zero16, res[j]) for j in range(NJ))

        return lax.fori_loop(lc_first, lc_last + 1, seg_body, accs_in)

    lax.fori_loop(0, nblk, blk_body, zeros48)

    def fin_group(q, carry):
        cbase = pl.multiple_of(c0 + q * 8, 8)

        @pl.when(cbase < C)
        def _():
            pltpu.sync_copy(ptab.at[pl.ds(cbase, 8), :], prow)

            def fin_row(rr, rcarry):
                lc = q * 8 + rr
                cnt = bnd[lc + 1] - bnd[lc]
                n = lax.convert_element_type(cnt, jnp.float32)
                nz = cnt > 0
                inv = _recip(jnp.maximum(n, 1.0))
                for j in range(NJ):
                    sm = acc[lc, pl.ds(j * L, L)]
                    pj = prow[rr, pl.ds(j * L, L)]
                    orow[rr, pl.ds(j * L, L)] = (
                        jnp.where(nz, sm * inv, 0.0) + pj)
                return rcarry

            lax.fori_loop(0, 8, fin_row, 0)
            pltpu.sync_copy(orow, out.at[pl.ds(cbase, 8), :])

        return carry

    lax.fori_loop(0, CPT // 8, fin_group, 0)


def kernel(embs, class_ids, personal_table, ln_gamma, ln_beta):
    ids32 = class_ids.astype(jnp.int32)
    yln = pl.pallas_call(
        _ln_body,
        grid=(N // 1024,),
        in_specs=[
            pl.BlockSpec((1024, D), lambda i: (i, 0)),
            pl.BlockSpec((1, D), lambda i: (0, 0)),
            pl.BlockSpec((1, D), lambda i: (0, 0)),
        ],
        out_specs=pl.BlockSpec((1024, D), lambda i: (i, 0)),
        out_shape=jax.ShapeDtypeStruct((N, D), jnp.float32),
    )(embs, ln_gamma.reshape(1, D), ln_beta.reshape(1, D))

    mesh = plsc.VectorSubcoreMesh(core_axis_name="c", subcore_axis_name="s")
    f = pl.kernel(
        _sc_body,
        out_type=jax.ShapeDtypeStruct((C, D), jnp.float32),
        mesh=mesh,
        scratch_types=[
            pltpu.VMEM((N,), jnp.int32),
            pltpu.VMEM((2, B, D), jnp.float32),
            pltpu.VMEM((CPT, D), jnp.float32),
            pltpu.SMEM((CPT + 1,), jnp.int32),
            pltpu.VMEM((8, D), jnp.float32),
            pltpu.VMEM((8, D), jnp.float32),
            pltpu.SemaphoreType.DMA,
        ],
    )
    return f(yln, ids32, personal_table)


# speedup vs baseline: 1.3797x; 1.3797x over previous
import jax
import jax.numpy as jnp
from jax import lax
from jax.experimental import pallas as pl
from jax.experimental.pallas import tpu as pltpu
from jax.experimental.pallas import tpu_sc as plsc

N = 16384
D = 768
C = 1000
L = 16
NJ = D // L
NC = 2
NS = 16
NW = NC * NS
CPT = 32
B = 32
EPS = 1e-5


def _recip(x):
    bits = lax.bitcast_convert_type(x, jnp.int32)
    r = lax.bitcast_convert_type(jnp.int32(0x7EF311C3) - bits, jnp.float32)
    for _ in range(4):
        r = r * (2.0 - x * r)
    return r


def _ln_body(x_ref, g_ref, b_ref, y_ref):
    x = x_ref[...]
    mean = jnp.mean(x, axis=1, keepdims=True)
    xc = x - mean
    var = jnp.mean(xc * xc, axis=1, keepdims=True)
    y_ref[...] = xc * lax.rsqrt(var + EPS) * g_ref[...] + b_ref[...]


def _sc_body(y, ids, ptab, out, ids_v, xb, acc, bnd, prow, orow, sem):
    wid = lax.axis_index("s") * NC + lax.axis_index("c")
    c0 = (wid * CPT).astype(jnp.int32)

    pltpu.sync_copy(ids, ids_v)

    def ids_at(g):
        base = g & ~(L - 1)
        v = ids_v[pl.ds(base, L)]
        off = g - base
        s = v[0]
        for k in range(1, L):
            s = jnp.where(off == k, v[k], s)
        return s

    NG = N // L

    def lower_bound(tgt):
        glo = jnp.int32(0)
        for sbit in range(10, -1, -1):
            cand = glo + jnp.int32(1 << sbit)
            probe = ids_v[pl.ds(jnp.minimum(cand, NG) * L - L, L)][L - 1]
            glo = jnp.where((cand <= NG) & (probe < tgt), cand, glo)
        base = jnp.minimum(glo, NG - 1) * L
        v = ids_v[pl.ds(base, L)]
        cnt = jnp.int32(0)
        for k in range(L):
            cnt = cnt + jnp.where(v[k] < tgt, 1, 0)
        return jnp.where(glo >= NG, jnp.int32(N), glo * L + cnt)

    zero16 = jnp.zeros((L,), jnp.float32)
    zeros48 = tuple(zero16 for _ in range(NJ))

    r0 = lower_bound(c0)
    r1 = lower_bound(c0 + CPT)
    bnd[0] = r0
    bnd[CPT] = r1
    start0 = r0 & ~7
    nblk = jnp.where(r1 > r0, (r1 - start0 + (B - 1)) >> 5, 0)

    def blk_base(k):
        return pl.multiple_of(jnp.minimum(start0 + k * B, N - B), 8)

    @pl.when(nblk > 0)
    def _():
        pltpu.async_copy(y.at[pl.ds(blk_base(0), B), :], xb.at[0], sem)

    def bnd_body(k, carry):
        bnd[k] = lower_bound(c0 + k)
        return carry

    lax.fori_loop(1, CPT, bnd_body, 0)

    def blk_body(k, accs_in):
        p = k & 1
        logical = start0 + k * B
        base = blk_base(k)
        pltpu.make_async_copy(y.at[pl.ds(base, B), :], xb.at[p], sem).wait()

        @pl.when(k + 1 < nblk)
        def _():
            pltpu.async_copy(
                y.at[pl.ds(blk_base(k + 1), B), :], xb.at[(k + 1) & 1], sem)

        i_lo = jnp.maximum(r0, logical) - base
        i_hi = jnp.minimum(r1, base + B) - base
        lc_first = ids_at(base + i_lo) - c0
        lc_last = ids_at(base + i_hi - 1) - c0

        def seg_body(lc, accs):
            first = lc == lc_first
            s_seg = jnp.maximum(bnd[lc], base + i_lo) - base
            e_seg = jnp.minimum(bnd[lc + 1], base + i_hi) - base
            init = tuple(
                jnp.where(first, accs[j], zero16) for j in range(NJ))

            npairs = (e_seg - s_seg) >> 1

            def pair_body(t, a):
                i = s_seg + t * 2
                return tuple(
                    a[j] + (xb[p, i, pl.ds(j * L, L)]
                            + xb[p, i + 1, pl.ds(j * L, L)])
                    for j in range(NJ))

            res = lax.fori_loop(0, npairs, pair_body, init)

            def row_body(i, a):
                return tuple(
                    a[j] + xb[p, i, pl.ds(j * L, L)] for j in range(NJ))

            res = lax.fori_loop(s_seg + npairs * 2, e_seg, row_body, res)
            closed = bnd[lc + 1] <= base + i_hi

            @pl.when(closed)
            def _():
                for j in range(NJ):
                    acc[lc, pl.ds(j * L, L)] = res[j]

            return tuple(
                jnp.where(closed, zero16, res[j]) for j in range(NJ))

        return lax.fori_loop(lc_first, lc_last + 1, seg_body, accs_in)

    lax.fori_loop(0, nblk, blk_body, zeros48)

    def fin_group(q, carry):
        cbase = pl.multiple_of(c0 + q * 8, 8)

        @pl.when(cbase < C)
        def _():
            pltpu.sync_copy(ptab.at[pl.ds(cbase, 8), :], prow)

            def fin_row(rr, rcarry):
                lc = q * 8 + rr
                cnt = bnd[lc + 1] - bnd[lc]
                n = lax.convert_element_type(cnt, jnp.float32)
                nz = cnt > 0
                inv = _recip(jnp.maximum(n, 1.0))
                for j in range(NJ):
                    sm = acc[lc, pl.ds(j * L, L)]
                    pj = prow[rr, pl.ds(j * L, L)]
                    orow[rr, pl.ds(j * L, L)] = (
                        jnp.where(nz, sm * inv, 0.0) + pj)
                return rcarry

            lax.fori_loop(0, 8, fin_row, 0)
            pltpu.sync_copy(orow, out.at[pl.ds(cbase, 8), :])

        return carry

    lax.fori_loop(0, CPT // 8, fin_group, 0)


def kernel(embs, class_ids, personal_table, ln_gamma, ln_beta):
    ids32 = class_ids.astype(jnp.int32)
    yln = pl.pallas_call(
        _ln_body,
        grid=(N // 512,),
        in_specs=[
            pl.BlockSpec((512, D), lambda i: (i, 0)),
            pl.BlockSpec((1, D), lambda i: (0, 0)),
            pl.BlockSpec((1, D), lambda i: (0, 0)),
        ],
        out_specs=pl.BlockSpec((512, D), lambda i: (i, 0)),
        out_shape=jax.ShapeDtypeStruct((N, D), jnp.float32),
    )(embs, ln_gamma.reshape(1, D), ln_beta.reshape(1, D))

    mesh = plsc.VectorSubcoreMesh(core_axis_name="c", subcore_axis_name="s")
    f = pl.kernel(
        _sc_body,
        out_type=jax.ShapeDtypeStruct((C, D), jnp.float32),
        mesh=mesh,
        scratch_types=[
            pltpu.VMEM((N,), jnp.int32),
            pltpu.VMEM((2, B, D), jnp.float32),
            pltpu.VMEM((CPT, D), jnp.float32),
            pltpu.SMEM((CPT + 1,), jnp.int32),
            pltpu.VMEM((8, D), jnp.float32),
            pltpu.VMEM((8, D), jnp.float32),
            pltpu.SemaphoreType.DMA,
        ],
    )
    return f(yln, ids32, personal_table)


# speedup vs baseline: 1.4832x; 1.0750x over previous
import jax
import jax.numpy as jnp
from jax import lax
from jax.experimental import pallas as pl
from jax.experimental.pallas import tpu as pltpu
from jax.experimental.pallas import tpu_sc as plsc

N = 16384
D = 768
C = 1000
L = 16
NJ = D // L
NC = 2
NS = 16
NW = NC * NS
CPT = 32
B = 32
EPS = 1e-5


def _recip(x):
    bits = lax.bitcast_convert_type(x, jnp.int32)
    r = lax.bitcast_convert_type(jnp.int32(0x7EF311C3) - bits, jnp.float32)
    for _ in range(4):
        r = r * (2.0 - x * r)
    return r


def _ln_body(x_ref, g_ref, b_ref, y_ref):
    x = x_ref[...]
    ones = jnp.ones((D, 128), jnp.float32)
    dn = (((1,), (0,)), ((), ()))
    s = lax.dot_general(x, ones, dn, preferred_element_type=jnp.float32)
    q = lax.dot_general(x * x, ones, dn, preferred_element_type=jnp.float32)
    mean = s[:, :1] * (1.0 / D)
    var = q[:, :1] * (1.0 / D) - mean * mean
    y_ref[...] = ((x - mean) * lax.rsqrt(var + EPS) * g_ref[...]
                  + b_ref[...])


def _sc_body(y, ids, ptab, out, ids_v, xb, acc, bnd, prow, orow, sem):
    wid = lax.axis_index("s") * NC + lax.axis_index("c")
    c0 = (wid * CPT).astype(jnp.int32)

    pltpu.sync_copy(ids, ids_v)

    def ids_at(g):
        base = g & ~(L - 1)
        v = ids_v[pl.ds(base, L)]
        off = g - base
        s = v[0]
        for k in range(1, L):
            s = jnp.where(off == k, v[k], s)
        return s

    NG = N // L

    def lower_bound(tgt):
        glo = jnp.int32(0)
        for sbit in range(10, -1, -1):
            cand = glo + jnp.int32(1 << sbit)
            probe = ids_v[pl.ds(jnp.minimum(cand, NG) * L - L, L)][L - 1]
            glo = jnp.where((cand <= NG) & (probe < tgt), cand, glo)
        base = jnp.minimum(glo, NG - 1) * L
        v = ids_v[pl.ds(base, L)]
        cnt = jnp.int32(0)
        for k in range(L):
            cnt = cnt + jnp.where(v[k] < tgt, 1, 0)
        return jnp.where(glo >= NG, jnp.int32(N), glo * L + cnt)

    zero16 = jnp.zeros((L,), jnp.float32)
    zeros48 = tuple(zero16 for _ in range(NJ))

    r0 = lower_bound(c0)
    r1 = lower_bound(c0 + CPT)
    bnd[0] = r0
    bnd[CPT] = r1
    start0 = r0 & ~7
    nblk = jnp.where(r1 > r0, (r1 - start0 + (B - 1)) >> 5, 0)

    def blk_base(k):
        return pl.multiple_of(jnp.minimum(start0 + k * B, N - B), 8)

    @pl.when(nblk > 0)
    def _():
        pltpu.async_copy(y.at[pl.ds(blk_base(0), B), :], xb.at[0], sem)

    def bnd_body(k, carry):
        bnd[k] = lower_bound(c0 + k)
        return carry

    lax.fori_loop(1, CPT, bnd_body, 0)

    def blk_body(k, accs_in):
        p = k & 1
        logical = start0 + k * B
        base = blk_base(k)
        pltpu.make_async_copy(y.at[pl.ds(base, B), :], xb.at[p], sem).wait()

        @pl.when(k + 1 < nblk)
        def _():
            pltpu.async_copy(
                y.at[pl.ds(blk_base(k + 1), B), :], xb.at[(k + 1) & 1], sem)

        i_lo = jnp.maximum(r0, logical) - base
        i_hi = jnp.minimum(r1, base + B) - base
        lc_first = ids_at(base + i_lo) - c0
        lc_last = ids_at(base + i_hi - 1) - c0

        def seg_body(lc, accs):
            first = lc == lc_first
            s_seg = jnp.maximum(bnd[lc], base + i_lo) - base
            e_seg = jnp.minimum(bnd[lc + 1], base + i_hi) - base
            init = tuple(
                jnp.where(first, accs[j], zero16) for j in range(NJ))

            def row_body(i, a):
                return tuple(
                    a[j] + xb[p, i, pl.ds(j * L, L)] for j in range(NJ))

            res = lax.fori_loop(s_seg, e_seg, row_body, init)
            closed = bnd[lc + 1] <= base + i_hi

            @pl.when(closed)
            def _():
                for j in range(NJ):
                    acc[lc, pl.ds(j * L, L)] = res[j]

            return tuple(
                jnp.where(closed, zero16, res[j]) for j in range(NJ))

        return lax.fori_loop(lc_first, lc_last + 1, seg_body, accs_in)

    lax.fori_loop(0, nblk, blk_body, zeros48)

    def fin_group(q, carry):
        cbase = pl.multiple_of(c0 + q * 8, 8)

        @pl.when(cbase < C)
        def _():
            pltpu.sync_copy(ptab.at[pl.ds(cbase, 8), :], prow)

            def fin_row(rr, rcarry):
                lc = q * 8 + rr
                cnt = bnd[lc + 1] - bnd[lc]
                n = lax.convert_element_type(cnt, jnp.float32)
                nz = cnt > 0
                inv = _recip(jnp.maximum(n, 1.0))
                for j in range(NJ):
                    sm = acc[lc, pl.ds(j * L, L)]
                    pj = prow[rr, pl.ds(j * L, L)]
                    orow[rr, pl.ds(j * L, L)] = (
                        jnp.where(nz, sm * inv, 0.0) + pj)
                return rcarry

            lax.fori_loop(0, 8, fin_row, 0)
            pltpu.sync_copy(orow, out.at[pl.ds(cbase, 8), :])

        return carry

    lax.fori_loop(0, CPT // 8, fin_group, 0)


def kernel(embs, class_ids, personal_table, ln_gamma, ln_beta):
    ids32 = class_ids.astype(jnp.int32)
    yln = pl.pallas_call(
        _ln_body,
        grid=(N // 512,),
        in_specs=[
            pl.BlockSpec((512, D), lambda i: (i, 0)),
            pl.BlockSpec((1, D), lambda i: (0, 0)),
            pl.BlockSpec((1, D), lambda i: (0, 0)),
        ],
        out_specs=pl.BlockSpec((512, D), lambda i: (i, 0)),
        out_shape=jax.ShapeDtypeStruct((N, D), jnp.float32),
    )(embs, ln_gamma.reshape(1, D), ln_beta.reshape(1, D))

    mesh = plsc.VectorSubcoreMesh(core_axis_name="c", subcore_axis_name="s")
    f = pl.kernel(
        _sc_body,
        out_type=jax.ShapeDtypeStruct((C, D), jnp.float32),
        mesh=mesh,
        scratch_types=[
            pltpu.VMEM((N,), jnp.int32),
            pltpu.VMEM((2, B, D), jnp.float32),
            pltpu.VMEM((CPT, D), jnp.float32),
            pltpu.SMEM((CPT + 1,), jnp.int32),
            pltpu.VMEM((8, D), jnp.float32),
            pltpu.VMEM((8, D), jnp.float32),
            pltpu.SemaphoreType.DMA,
        ],
    )
    return f(yln, ids32, personal_table)
